# baseline (speedup 1.0000x reference)
import functools
import math

import jax
import jax.numpy as jnp
from jax import lax
from jax.experimental import pallas as pl
from jax.experimental.pallas import tpu as pltpu
from jax.experimental.pallas import tpu_sc as plsc

B, N1, N2 = 4, 4096, 4096
TI = 512
NBLK = N1 // TI
BN1 = B * N1


def _acos01(x):
    p = jnp.float32(-0.0012624911)
    for c in (0.0066700901, -0.0170881256, 0.0308918810, -0.0501743046,
              0.0889789874, -0.2145988016, 1.5707963050):
        p = p * x + jnp.float32(c)
    return p * jnp.sqrt(jnp.maximum(1.0 - x, 0.0))


def _nn_body(x1_ref, x2_ref, x2t_ref, idx_ref):
    b = pl.program_id(0)
    x1 = x1_ref[0, 0]
    x2 = x2_ref[0]

    dotsm2 = jax.lax.dot_general(
        x1 * -2.0, x2, (((1,), (1,)), ((), ())), preferred_element_type=jnp.float32
    )
    r0 = x2t_ref[0, 0:1, :]
    r1 = x2t_ref[0, 1:2, :]
    r2 = x2t_ref[0, 2:3, :]
    sq1 = jnp.sum(x1 * x1, axis=1, keepdims=True)
    sq2 = (r0 * r0 + r1 * r1) + r2 * r2

    CH = 128
    d0 = (sq1 + sq2[:, 0:CH]) + dotsm2[:, 0:CH]
    runv = d0
    runi = jnp.zeros((TI, CH), jnp.int32)
    for g in range(1, N2 // CH):
        dg = (sq1 + sq2[:, g * CH:(g + 1) * CH]) + dotsm2[:, g * CH:(g + 1) * CH]
        c = dg < runv
        runv = jnp.where(c, dg, runv)
        runi = jnp.where(c, jnp.int32(g), runi)

    lane = lax.broadcasted_iota(jnp.int32, (TI, CH), 1)
    jfull = runi * CH + lane
    dmin = jnp.min(runv, axis=1, keepdims=True)
    idx = jnp.min(jnp.where(runv <= dmin, jfull, N2), axis=1, keepdims=True)
    idx_ref[0, 0] = idx + b * N2


def _finish_body(xn_ref, g6_ref, dist_ref, ang_ref):
    x1t_ref = xn_ref
    x10 = x1t_ref[0:1, 0:BN1]
    x11 = x1t_ref[1:2, 0:BN1]
    x12 = x1t_ref[2:3, 0:BN1]
    p0 = g6_ref[0:1, :]
    p1 = g6_ref[1:2, :]
    p2 = g6_ref[2:3, :]
    g0 = g6_ref[3:4, :]
    g1 = g6_ref[4:5, :]
    g2 = g6_ref[5:6, :]

    d0, d1, d2 = x10 - p0, x11 - p1, x12 - p2
    dist_ref[0:1, :] = (d0 * d0 + d1 * d1) + d2 * d2

    m0 = xn_ref[0:1, BN1:2 * BN1]
    m1 = xn_ref[1:2, BN1:2 * BN1]
    m2 = xn_ref[2:3, BN1:2 * BN1]
    n1n = jnp.sqrt((m0 * m0 + m1 * m1) + m2 * m2)
    inv1 = 1.0 / jnp.maximum(n1n, 1e-12)
    n2n = jnp.sqrt((g0 * g0 + g1 * g1) + g2 * g2)
    inv2 = 1.0 / jnp.maximum(n2n, 1e-12)
    dotn = (m0 * g0 + m1 * g1) + m2 * g2
    cosang = jnp.abs(dotn * inv1 * inv2)
    angle = _acos01(jnp.clip(cosang, 0.0, 1.0)) * (180.0 / math.pi)
    ang_ref[0:1, :] = angle


def _make_sc_gather():
    info = plsc.get_sparse_core_info()
    NC, NS, L = info.num_cores, info.num_subcores, info.num_lanes
    NW = NC * NS
    b_per_w = BN1 // NW
    nchunk = b_per_w // L
    mesh = plsc.VectorSubcoreMesh(core_axis_name="c", subcore_axis_name="s")

    @functools.partial(
        pl.kernel, mesh=mesh,
        compiler_params=pltpu.CompilerParams(needs_layout_passes=False),
        out_type=jax.ShapeDtypeStruct((6, BN1), jnp.float32),
        scratch_types=[
            pltpu.VMEM((b_per_w,), jnp.int32),
            pltpu.VMEM((6, b_per_w), jnp.int32),
            pltpu.VMEM((6, b_per_w), jnp.float32),
            pltpu.SemaphoreType.DMA,
        ],
    )
    def gather_k(table_hbm, idx_hbm, out_hbm, idx_v, fvk_v, out_v, sem):
        wid = lax.axis_index("s") * NC + lax.axis_index("c")
        base = wid * b_per_w
        pltpu.sync_copy(idx_hbm.at[pl.ds(base, b_per_w)], idx_v)
        for c in range(nchunk):
            iv = idx_v[pl.ds(c * L, L)]
            fv = iv * 6
            for k in range(6):
                fvk_v[k, pl.ds(c * L, L)] = fv + k
        W = 128
        copies = []
        for k in range(6):
            for w in range(b_per_w // W):
                copies.append(pltpu.async_copy(
                    table_hbm.at[fvk_v.at[k, pl.ds(w * W, W)]],
                    out_v.at[k, pl.ds(w * W, W)],
                    sem,
                ))
        for cp in copies:
            cp.wait()
        for k in range(6):
            pltpu.sync_copy(out_v.at[k], out_hbm.at[k, pl.ds(base, b_per_w)])

    return gather_k


_sc_gather = _make_sc_gather()


def kernel(xyz1, xyz2, normal_rebuild, normal_gt):
    x1r = xyz1.reshape(B, NBLK, TI, 3)
    x2t = jnp.transpose(xyz2, (0, 2, 1))

    idxg = pl.pallas_call(
        _nn_body,
        grid=(B, NBLK),
        in_specs=[
            pl.BlockSpec((1, 1, TI, 3), lambda b, i: (b, i, 0, 0)),
            pl.BlockSpec((1, N2, 3), lambda b, i: (b, 0, 0)),
            pl.BlockSpec((1, 3, N2), lambda b, i: (b, 0, 0)),
        ],
        out_specs=pl.BlockSpec((1, 1, TI, 1), lambda b, i: (b, i, 0, 0)),
        out_shape=jax.ShapeDtypeStruct((B, NBLK, TI, 1), jnp.int32),
    )(x1r, xyz2, x2t)

    table = jnp.concatenate([xyz2, normal_gt], axis=2).reshape(B * N2 * 6)
    g6 = _sc_gather(table, idxg.reshape(BN1))

    xn = jnp.transpose(
        jnp.concatenate(
            [xyz1.reshape(BN1, 3), normal_rebuild.reshape(BN1, 3)], axis=0
        ),
        (1, 0),
    )

    dist, ang = pl.pallas_call(
        _finish_body,
        grid=(1,),
        in_specs=[
            pl.BlockSpec((3, 2 * BN1), lambda i: (0, 0)),
            pl.BlockSpec((6, BN1), lambda i: (0, 0)),
        ],
        out_specs=(
            pl.BlockSpec((1, BN1), lambda i: (0, 0)),
            pl.BlockSpec((1, BN1), lambda i: (0, 0)),
        ),
        out_shape=(
            jax.ShapeDtypeStruct((1, BN1), jnp.float32),
            jax.ShapeDtypeStruct((1, BN1), jnp.float32),
        ),
    )(xn, g6)

    return (
        jnp.transpose(g6[0:3, :], (1, 0)).reshape(B, N1, 3),
        jnp.transpose(g6[3:6, :], (1, 0)).reshape(B, N1, 3),
        dist.reshape(B, N1),
        ang.reshape(B, N1),
    )


# speedup vs baseline: 1.0566x; 1.0566x over previous
import functools
import math

import jax
import jax.numpy as jnp
from jax import lax
from jax.experimental import pallas as pl
from jax.experimental.pallas import tpu as pltpu
from jax.experimental.pallas import tpu_sc as plsc

B, N1, N2 = 4, 4096, 4096
TI = 1024
NBLK = N1 // TI
BN1 = B * N1


def _acos01(x):
    p = jnp.float32(-0.0012624911)
    for c in (0.0066700901, -0.0170881256, 0.0308918810, -0.0501743046,
              0.0889789874, -0.2145988016, 1.5707963050):
        p = p * x + jnp.float32(c)
    return p * jnp.sqrt(jnp.maximum(1.0 - x, 0.0))


def _nn_body(x1_ref, x2_ref, x2t_ref, idx_ref):
    b = pl.program_id(0)
    x1 = x1_ref[0, 0]
    x2 = x2_ref[0]

    dotsm2 = jax.lax.dot_general(
        x1 * -2.0, x2, (((1,), (1,)), ((), ())), preferred_element_type=jnp.float32
    )
    r0 = x2t_ref[0, 0:1, :]
    r1 = x2t_ref[0, 1:2, :]
    r2 = x2t_ref[0, 2:3, :]
    sq1 = jnp.sum(x1 * x1, axis=1, keepdims=True)
    sq2 = (r0 * r0 + r1 * r1) + r2 * r2

    CH = 128
    d0 = (sq1 + sq2[:, 0:CH]) + dotsm2[:, 0:CH]
    runv = d0
    runi = jnp.zeros((TI, CH), jnp.int32)
    for g in range(1, N2 // CH):
        dg = (sq1 + sq2[:, g * CH:(g + 1) * CH]) + dotsm2[:, g * CH:(g + 1) * CH]
        c = dg < runv
        runv = jnp.where(c, dg, runv)
        runi = jnp.where(c, jnp.int32(g), runi)

    lane = lax.broadcasted_iota(jnp.int32, (TI, CH), 1)
    jfull = runi * CH + lane
    dmin = jnp.min(runv, axis=1, keepdims=True)
    idx = jnp.min(jnp.where(runv <= dmin, jfull, N2), axis=1, keepdims=True)
    idx_ref[0, 0] = idx + b * N2


def _finish_body(xn_ref, g6_ref, dist_ref, ang_ref):
    x1t_ref = xn_ref
    x10 = x1t_ref[0:1, 0:BN1]
    x11 = x1t_ref[1:2, 0:BN1]
    x12 = x1t_ref[2:3, 0:BN1]
    p0 = g6_ref[0:1, :]
    p1 = g6_ref[1:2, :]
    p2 = g6_ref[2:3, :]
    g0 = g6_ref[3:4, :]
    g1 = g6_ref[4:5, :]
    g2 = g6_ref[5:6, :]

    d0, d1, d2 = x10 - p0, x11 - p1, x12 - p2
    dist_ref[0:1, :] = (d0 * d0 + d1 * d1) + d2 * d2

    m0 = xn_ref[0:1, BN1:2 * BN1]
    m1 = xn_ref[1:2, BN1:2 * BN1]
    m2 = xn_ref[2:3, BN1:2 * BN1]
    n1n = jnp.sqrt((m0 * m0 + m1 * m1) + m2 * m2)
    inv1 = 1.0 / jnp.maximum(n1n, 1e-12)
    n2n = jnp.sqrt((g0 * g0 + g1 * g1) + g2 * g2)
    inv2 = 1.0 / jnp.maximum(n2n, 1e-12)
    dotn = (m0 * g0 + m1 * g1) + m2 * g2
    cosang = jnp.abs(dotn * inv1 * inv2)
    angle = _acos01(jnp.clip(cosang, 0.0, 1.0)) * (180.0 / math.pi)
    ang_ref[0:1, :] = angle


def _make_sc_gather():
    info = plsc.get_sparse_core_info()
    NC, NS, L = info.num_cores, info.num_subcores, info.num_lanes
    NW = NC * NS
    b_per_w = BN1 // NW
    nchunk = b_per_w // L
    mesh = plsc.VectorSubcoreMesh(core_axis_name="c", subcore_axis_name="s")

    @functools.partial(
        pl.kernel, mesh=mesh,
        compiler_params=pltpu.CompilerParams(needs_layout_passes=False),
        out_type=jax.ShapeDtypeStruct((6, BN1), jnp.float32),
        scratch_types=[
            pltpu.VMEM((b_per_w,), jnp.int32),
            pltpu.VMEM((6, b_per_w), jnp.int32),
            pltpu.VMEM((6, b_per_w), jnp.float32),
            pltpu.SemaphoreType.DMA,
        ],
    )
    def gather_k(table_hbm, idx_hbm, out_hbm, idx_v, fvk_v, out_v, sem):
        wid = lax.axis_index("s") * NC + lax.axis_index("c")
        base = wid * b_per_w
        pltpu.sync_copy(idx_hbm.at[pl.ds(base, b_per_w)], idx_v)
        for c in range(nchunk):
            iv = idx_v[pl.ds(c * L, L)]
            fv = iv * 6
            for k in range(6):
                fvk_v[k, pl.ds(c * L, L)] = fv + k
        W = 128
        copies = []
        for k in range(6):
            for w in range(b_per_w // W):
                copies.append(pltpu.async_copy(
                    table_hbm.at[fvk_v.at[k, pl.ds(w * W, W)]],
                    out_v.at[k, pl.ds(w * W, W)],
                    sem,
                ))
        for cp in copies:
            cp.wait()
        for k in range(6):
            pltpu.sync_copy(out_v.at[k], out_hbm.at[k, pl.ds(base, b_per_w)])

    return gather_k


_sc_gather = _make_sc_gather()


def kernel(xyz1, xyz2, normal_rebuild, normal_gt):
    x1r = xyz1.reshape(B, NBLK, TI, 3)
    x2t = jnp.transpose(xyz2, (0, 2, 1))

    idxg = pl.pallas_call(
        _nn_body,
        grid=(B, NBLK),
        in_specs=[
            pl.BlockSpec((1, 1, TI, 3), lambda b, i: (b, i, 0, 0)),
            pl.BlockSpec((1, N2, 3), lambda b, i: (b, 0, 0)),
            pl.BlockSpec((1, 3, N2), lambda b, i: (b, 0, 0)),
        ],
        out_specs=pl.BlockSpec((1, 1, TI, 1), lambda b, i: (b, i, 0, 0)),
        out_shape=jax.ShapeDtypeStruct((B, NBLK, TI, 1), jnp.int32),
    )(x1r, xyz2, x2t)

    table = jnp.concatenate([xyz2, normal_gt], axis=2).reshape(B * N2 * 6)
    g6 = _sc_gather(table, idxg.reshape(BN1))

    xn = jnp.transpose(
        jnp.concatenate(
            [xyz1.reshape(BN1, 3), normal_rebuild.reshape(BN1, 3)], axis=0
        ),
        (1, 0),
    )

    dist, ang = pl.pallas_call(
        _finish_body,
        grid=(1,),
        in_specs=[
            pl.BlockSpec((3, 2 * BN1), lambda i: (0, 0)),
            pl.BlockSpec((6, BN1), lambda i: (0, 0)),
        ],
        out_specs=(
            pl.BlockSpec((1, BN1), lambda i: (0, 0)),
            pl.BlockSpec((1, BN1), lambda i: (0, 0)),
        ),
        out_shape=(
            jax.ShapeDtypeStruct((1, BN1), jnp.float32),
            jax.ShapeDtypeStruct((1, BN1), jnp.float32),
        ),
    )(xn, g6)

    return (
        jnp.transpose(g6[0:3, :], (1, 0)).reshape(B, N1, 3),
        jnp.transpose(g6[3:6, :], (1, 0)).reshape(B, N1, 3),
        dist.reshape(B, N1),
        ang.reshape(B, N1),
    )


# speedup vs baseline: 1.0800x; 1.0222x over previous
import functools
import math

import jax
import jax.numpy as jnp
from jax import lax
from jax.experimental import pallas as pl
from jax.experimental.pallas import tpu as pltpu
from jax.experimental.pallas import tpu_sc as plsc

B, N1, N2 = 4, 4096, 4096
TI = 2048
NBLK = N1 // TI
BN1 = B * N1


def _acos01(x):
    p = jnp.float32(-0.0012624911)
    for c in (0.0066700901, -0.0170881256, 0.0308918810, -0.0501743046,
              0.0889789874, -0.2145988016, 1.5707963050):
        p = p * x + jnp.float32(c)
    return p * jnp.sqrt(jnp.maximum(1.0 - x, 0.0))


def _nn_body(x1_ref, x2_ref, x2t_ref, idx_ref):
    b = pl.program_id(0)
    x1 = x1_ref[0, 0]
    x2 = x2_ref[0]

    dotsm2 = jax.lax.dot_general(
        x1 * -2.0, x2, (((1,), (1,)), ((), ())), preferred_element_type=jnp.float32
    )
    r0 = x2t_ref[0, 0:1, :]
    r1 = x2t_ref[0, 1:2, :]
    r2 = x2t_ref[0, 2:3, :]
    sq1 = jnp.sum(x1 * x1, axis=1, keepdims=True)
    sq2 = (r0 * r0 + r1 * r1) + r2 * r2

    CH = 128
    d0 = (sq1 + sq2[:, 0:CH]) + dotsm2[:, 0:CH]
    runv = d0
    runi = jnp.zeros((TI, CH), jnp.int32)
    for g in range(1, N2 // CH):
        dg = (sq1 + sq2[:, g * CH:(g + 1) * CH]) + dotsm2[:, g * CH:(g + 1) * CH]
        c = dg < runv
        runv = jnp.where(c, dg, runv)
        runi = jnp.where(c, jnp.int32(g), runi)

    lane = lax.broadcasted_iota(jnp.int32, (TI, CH), 1)
    jfull = runi * CH + lane
    dmin = jnp.min(runv, axis=1, keepdims=True)
    idx = jnp.min(jnp.where(runv <= dmin, jfull, N2), axis=1, keepdims=True)
    idx_ref[0, 0] = idx + b * N2


def _finish_body(xn_ref, g6_ref, dist_ref, ang_ref):
    x1t_ref = xn_ref
    x10 = x1t_ref[0:1, 0:BN1]
    x11 = x1t_ref[1:2, 0:BN1]
    x12 = x1t_ref[2:3, 0:BN1]
    p0 = g6_ref[0:1, :]
    p1 = g6_ref[1:2, :]
    p2 = g6_ref[2:3, :]
    g0 = g6_ref[3:4, :]
    g1 = g6_ref[4:5, :]
    g2 = g6_ref[5:6, :]

    d0, d1, d2 = x10 - p0, x11 - p1, x12 - p2
    dist_ref[0:1, :] = (d0 * d0 + d1 * d1) + d2 * d2

    m0 = xn_ref[0:1, BN1:2 * BN1]
    m1 = xn_ref[1:2, BN1:2 * BN1]
    m2 = xn_ref[2:3, BN1:2 * BN1]
    n1n = jnp.sqrt((m0 * m0 + m1 * m1) + m2 * m2)
    inv1 = 1.0 / jnp.maximum(n1n, 1e-12)
    n2n = jnp.sqrt((g0 * g0 + g1 * g1) + g2 * g2)
    inv2 = 1.0 / jnp.maximum(n2n, 1e-12)
    dotn = (m0 * g0 + m1 * g1) + m2 * g2
    cosang = jnp.abs(dotn * inv1 * inv2)
    angle = _acos01(jnp.clip(cosang, 0.0, 1.0)) * (180.0 / math.pi)
    ang_ref[0:1, :] = angle


def _make_sc_gather():
    info = plsc.get_sparse_core_info()
    NC, NS, L = info.num_cores, info.num_subcores, info.num_lanes
    NW = NC * NS
    b_per_w = BN1 // NW
    nchunk = b_per_w // L
    mesh = plsc.VectorSubcoreMesh(core_axis_name="c", subcore_axis_name="s")

    @functools.partial(
        pl.kernel, mesh=mesh,
        compiler_params=pltpu.CompilerParams(needs_layout_passes=False),
        out_type=jax.ShapeDtypeStruct((6, BN1), jnp.float32),
        scratch_types=[
            pltpu.VMEM((b_per_w,), jnp.int32),
            pltpu.VMEM((6, b_per_w), jnp.int32),
            pltpu.VMEM((6, b_per_w), jnp.float32),
            pltpu.SemaphoreType.DMA,
        ],
    )
    def gather_k(table_hbm, idx_hbm, out_hbm, idx_v, fvk_v, out_v, sem):
        wid = lax.axis_index("s") * NC + lax.axis_index("c")
        base = wid * b_per_w
        pltpu.sync_copy(idx_hbm.at[pl.ds(base, b_per_w)], idx_v)
        for c in range(nchunk):
            iv = idx_v[pl.ds(c * L, L)]
            fv = iv * 6
            for k in range(6):
                fvk_v[k, pl.ds(c * L, L)] = fv + k
        W = 128
        copies = []
        for k in range(6):
            for w in range(b_per_w // W):
                copies.append(pltpu.async_copy(
                    table_hbm.at[fvk_v.at[k, pl.ds(w * W, W)]],
                    out_v.at[k, pl.ds(w * W, W)],
                    sem,
                ))
        for cp in copies:
            cp.wait()
        for k in range(6):
            pltpu.sync_copy(out_v.at[k], out_hbm.at[k, pl.ds(base, b_per_w)])

    return gather_k


_sc_gather = _make_sc_gather()


def kernel(xyz1, xyz2, normal_rebuild, normal_gt):
    x1r = xyz1.reshape(B, NBLK, TI, 3)
    x2t = jnp.transpose(xyz2, (0, 2, 1))

    idxg = pl.pallas_call(
        _nn_body,
        grid=(B, NBLK),
        in_specs=[
            pl.BlockSpec((1, 1, TI, 3), lambda b, i: (b, i, 0, 0)),
            pl.BlockSpec((1, N2, 3), lambda b, i: (b, 0, 0)),
            pl.BlockSpec((1, 3, N2), lambda b, i: (b, 0, 0)),
        ],
        out_specs=pl.BlockSpec((1, 1, TI, 1), lambda b, i: (b, i, 0, 0)),
        out_shape=jax.ShapeDtypeStruct((B, NBLK, TI, 1), jnp.int32),
    )(x1r, xyz2, x2t)

    table = jnp.concatenate([xyz2, normal_gt], axis=2).reshape(B * N2 * 6)
    g6 = _sc_gather(table, idxg.reshape(BN1))

    xn = jnp.transpose(
        jnp.concatenate(
            [xyz1.reshape(BN1, 3), normal_rebuild.reshape(BN1, 3)], axis=0
        ),
        (1, 0),
    )

    dist, ang = pl.pallas_call(
        _finish_body,
        grid=(1,),
        in_specs=[
            pl.BlockSpec((3, 2 * BN1), lambda i: (0, 0)),
            pl.BlockSpec((6, BN1), lambda i: (0, 0)),
        ],
        out_specs=(
            pl.BlockSpec((1, BN1), lambda i: (0, 0)),
            pl.BlockSpec((1, BN1), lambda i: (0, 0)),
        ),
        out_shape=(
            jax.ShapeDtypeStruct((1, BN1), jnp.float32),
            jax.ShapeDtypeStruct((1, BN1), jnp.float32),
        ),
    )(xn, g6)

    return (
        jnp.transpose(g6[0:3, :], (1, 0)).reshape(B, N1, 3),
        jnp.transpose(g6[3:6, :], (1, 0)).reshape(B, N1, 3),
        dist.reshape(B, N1),
        ang.reshape(B, N1),
    )
